# baseline (speedup 1.0000x reference)
import jax
import jax.numpy as jnp
from jax.experimental import pallas as pl
from jax.experimental.pallas import tpu as pltpu

_VOCAB = 1000
_SEQ = 50
_BATCH = 1024
_BLK = 32
_N = _BATCH // _BLK
_NSLOT = 4
_NRATINGS = (_SEQ - 1) // 2


def _avg_onehot(xb):
    blk = xb.shape[0]
    col = jax.lax.broadcasted_iota(jnp.int32, (blk, _SEQ), 1)
    mask = (col >= 2) & (col % 2 == 0)
    s = jnp.sum(jnp.where(mask, xb.astype(jnp.float32), 0.0), axis=1)
    s = s.astype(jnp.int32)
    q = s // _NRATINGS
    r = s - q * _NRATINGS
    half = _NRATINGS // 2
    inc = (r > half) | ((r == half) & ((q & 1) == 1))
    avg = q + inc.astype(jnp.int32)
    voc = jax.lax.broadcasted_iota(jnp.int32, (blk, _VOCAB), 1)
    return (voc == avg[:, None]).astype(jnp.float32)


def _body(x_ref, o_ref, vmem, sem):
    i = pl.program_id(0)
    slot = jax.lax.rem(i, _NSLOT)

    @pl.when(i >= _NSLOT)
    def _wait_prev():
        prev = i - _NSLOT
        pltpu.make_async_copy(
            vmem.at[slot], o_ref.at[pl.ds(prev * _BLK, _BLK)], sem.at[slot]
        ).wait()

    xb = x_ref[pl.ds(i * _BLK, _BLK), :]
    onehot = _avg_onehot(xb)
    vmem[slot] = jnp.zeros((_BLK, _SEQ, _VOCAB), jnp.float32)
    vmem[slot, :, _SEQ - 1 : _SEQ, :] = onehot[:, None, :]
    pltpu.make_async_copy(
        vmem.at[slot], o_ref.at[pl.ds(i * _BLK, _BLK)], sem.at[slot]
    ).start()

    @pl.when(i == _N - 1)
    def _drain():
        for j in range(_NSLOT - 1, -1, -1):
            step = _N - 1 - j
            if step >= 0:
                s2 = step % _NSLOT
                pltpu.make_async_copy(
                    vmem.at[s2], o_ref.at[pl.ds(step * _BLK, _BLK)], sem.at[s2]
                ).wait()


def kernel(x):
    return pl.pallas_call(
        _body,
        grid=(_N,),
        in_specs=[pl.BlockSpec((_BATCH, _SEQ), lambda i: (0, 0))],
        out_specs=pl.BlockSpec(memory_space=pl.ANY),
        out_shape=jax.ShapeDtypeStruct((_BATCH, _SEQ, _VOCAB), jnp.float32),
        scratch_shapes=[
            pltpu.VMEM((_NSLOT, _BLK, _SEQ, _VOCAB), jnp.float32),
            pltpu.SemaphoreType.DMA((_NSLOT,)),
        ],
        compiler_params=pltpu.CompilerParams(
            dimension_semantics=("arbitrary",),
        ),
    )(x)
